# baseline (speedup 1.0000x reference)
import jax
import jax.numpy as jnp
import numpy as np
from jax import lax
from jax.experimental import pallas as pl
from jax.experimental.pallas import tpu as pltpu
from jax.experimental.pallas import tpu_sc as plsc

_N = 10000
_E = 160000
_H = 128
_R = 32
_CUT = 5.0

_BN = 1000
_BE = 2000

_BEM = 2048
_TBLK = 44
_TE = _TBLK * _BEM
_NW = 32
_BSC = 32
_NBB = 70
_EPW = _BSC * _NBB
_EP = _TE + _NW * _EPW
_NPT = 624
_NTAIL = _N - 16 * _NPT
_ZR = 16
_G_OF_K = (0, 1, 1, 1, 2, 2, 2, 2, 2)


def _silu(x):
    return x / (1.0 + jnp.exp(-x))


def _edge_mlp_body(attr_ref, ew_ref, ws1, b1, ws2, b2,
                   w3a, b3a, w3b, b3b, w3c, b3c, out_ref):
    x = attr_ref[...]
    h1 = _silu(jnp.dot(x, ws1[...], preferred_element_type=jnp.float32) + b1[...])
    h2 = _silu(jnp.dot(h1, ws2[...], preferred_element_type=jnp.float32) + b2[...])
    r = ew_ref[...]
    c = 0.5 * (jnp.cos(r * (np.pi / _CUT)) + 1.0) * (r < _CUT).astype(jnp.float32)
    for ci, (w, b) in enumerate(((w3a, b3a), (w3b, b3b), (w3c, b3c))):
        out_ref[ci] = _silu(
            jnp.dot(h2, w[...], preferred_element_type=jnp.float32) + b[...]) * c


def _prep_body(x_ref, wt0, wt1, wt2, xn_ref, v_ref, va_ref, vb_ref):
    x = x_ref[...]
    nrm = (x * x).sum(axis=0)
    xn = x / (nrm + 1.0)
    xn_ref[...] = xn
    iv = (xn[0] + xn[4] + xn[8]) * (1.0 / 3.0)
    a01 = 0.5 * (xn[1] - xn[3])
    a02 = 0.5 * (xn[2] - xn[6])
    a12 = 0.5 * (xn[5] - xn[7])
    s00 = xn[0] - iv
    s11 = xn[4] - iv
    s01 = 0.5 * (xn[1] + xn[3])
    s02 = 0.5 * (xn[2] + xn[6])
    s12 = 0.5 * (xn[5] + xn[7])
    w0 = wt0[...]
    w1 = wt1[...]
    w2 = wt2[...]
    dot = lambda a, w: jnp.dot(a, w, preferred_element_type=jnp.float32)
    planes = (dot(iv, w0), dot(a01, w1), dot(a02, w1), dot(a12, w1),
              dot(s00, w2), dot(s01, w2), dot(s02, w2),
              dot(s11, w2), dot(s12, w2))
    for k in range(9):
        v_ref[k] = planes[k]
    for k in range(4):
        va_ref[:, k * _H:(k + 1) * _H] = planes[k]
    for k in range(5):
        vb_ref[:, k * _H:(k + 1) * _H] = planes[4 + k]


def _make_msg_body(comp_of_k):
    def _msg_body(src_ref, dst_ref, ea_ref, v_ref, msg_ref):
        @pl.when(pl.program_id(0) == 0)
        def _():
            msg_ref[...] = jnp.zeros_like(msg_ref)

        ncomp = len(sorted(set(comp_of_k)))

        def body(i, carry):
            s = src_ref[0, 0, i]
            d = dst_ref[0, 0, i]
            es = [ea_ref[c, pl.ds(i, 1), :] for c in range(ncomp)]
            m = jnp.concatenate([es[c] for c in comp_of_k], axis=1)
            msg_ref[pl.ds(d, 1), :] += v_ref[pl.ds(s, 1), :] * m
            return carry

        jax.lax.fori_loop(0, _BEM, body, 0, unroll=2)

    return _msg_body


def _sc_msg_kernel(v_hbm, ea_hbm, idx_hbm, out_hbm,
                   idxb0, idxb1, rows0, rows1, eav0, eav1, zrow, shared,
                   si0, si1, sg0, sg1, se0, se1):
    cid = lax.axis_index("c")
    sid = lax.axis_index("s")
    wid = cid * 16 + sid
    edge_base = _TE + wid * _EPW
    row_lo = sid * _NPT

    def zinit(i, c):
        for j in range(_H // 16):
            zrow[i, pl.ds(j * 16, 16)] = jnp.zeros((16,), jnp.float32)
        return c

    lax.fori_loop(0, _ZR, zinit, 0)

    def start_idx(b, idxb, si):
        pltpu.async_copy(idx_hbm.at[wid, b], idxb, si)

    def wait_idx(b, idxb, si):
        pltpu.make_async_copy(idx_hbm.at[wid, b], idxb, si).wait()

    for k in range(9):
        gk = _G_OF_K[k]

        def start_ge(b, idxb, rows, eav, sg, se):
            base = pl.multiple_of(edge_base + b * _BSC, _BSC)
            pltpu.async_copy(v_hbm.at[k].at[idxb.at[0]], rows, sg)
            pltpu.async_copy(ea_hbm.at[gk, pl.ds(base, _BSC)], eav, se)

        def finish(b, idxb, rows, eav, sg, se):
            pltpu.make_async_copy(v_hbm.at[k].at[idxb.at[0]],
                                  rows, sg).wait()
            base = pl.multiple_of(edge_base + b * _BSC, _BSC)
            pltpu.make_async_copy(ea_hbm.at[gk, pl.ds(base, _BSC)],
                                  eav, se).wait()

            def edge(i, c2):
                for j in range(_H // 16):
                    rows[i, pl.ds(j * 16, 16)] = (
                        rows[i, pl.ds(j * 16, 16)]
                        * eav[i, pl.ds(j * 16, 16)])
                return c2

            lax.fori_loop(0, _BSC, edge, 0, unroll=2)
            pltpu.sync_copy(rows, shared.at[idxb.at[1]], add=True)

        for z in range(_NPT // _ZR):
            pltpu.sync_copy(zrow, shared.at[pl.ds(row_lo + z * _ZR, _ZR)])

        @pl.when(sid == 15)
        def _():
            pltpu.sync_copy(zrow.at[pl.ds(0, _NTAIL)],
                            shared.at[pl.ds(16 * _NPT, _NTAIL)])

        plsc.subcore_barrier()
        pltpu.sync_copy(idx_hbm.at[wid, 0], idxb0)
        start_ge(0, idxb0, rows0, eav0, sg0, se0)
        start_idx(1, idxb1, si1)

        def pair(p, carry):
            b0 = p * 2
            wait_idx(b0 + 1, idxb1, si1)
            start_ge(b0 + 1, idxb1, rows1, eav1, sg1, se1)
            finish(b0, idxb0, rows0, eav0, sg0, se0)

            @pl.when(b0 + 2 < _NBB)
            def _():
                start_idx(b0 + 2, idxb0, si0)

            finish(b0 + 1, idxb1, rows1, eav1, sg1, se1)

            @pl.when(b0 + 3 < _NBB)
            def _():
                start_idx(b0 + 3, idxb1, si1)

            @pl.when(b0 + 2 < _NBB)
            def _():
                wait_idx(b0 + 2, idxb0, si0)
                start_ge(b0 + 2, idxb0, rows0, eav0, sg0, se0)

            return carry

        lax.fori_loop(0, _NBB // 2, pair, 0)
        plsc.subcore_barrier()
        pltpu.sync_copy(shared.at[pl.ds(row_lo, _NPT)],
                        out_hbm.at[cid, k, pl.ds(row_lo, _NPT)])

        @pl.when(sid == 15)
        def _():
            pltpu.sync_copy(shared.at[pl.ds(16 * _NPT, _NTAIL)],
                            out_hbm.at[cid, k, pl.ds(16 * _NPT, _NTAIL)])

        plsc.subcore_barrier()


def _full9(t):
    iv, a01, a02, a12, s00, s01, s02, s11, s12 = t
    return (iv + s00, s01 + a01, s02 + a02,
            s01 - a01, iv + s11, s12 + a12,
            s02 - a02, s12 - a12, iv - s00 - s11)


def _post_body(xn_ref, v_ref, m_ref, ma_ref, mb_ref, wt3, wt4, wt5, out_ref):
    vv = v_ref[...]
    vm = m_ref[0] + m_ref[1]
    Y = _full9(tuple(vv[k] for k in range(9)))
    M = _full9(tuple(
        vm[k] + (ma_ref[:, k * _H:(k + 1) * _H] if k < 4
                 else mb_ref[:, (k - 4) * _H:(k - 3) * _H])
        for k in range(9)))
    y = [[Y[0], Y[1], Y[2]], [Y[3], Y[4], Y[5]], [Y[6], Y[7], Y[8]]]
    m = [[M[0], M[1], M[2]], [M[3], M[4], M[5]], [M[6], M[7], M[8]]]
    c = [[None] * 3 for _ in range(3)]
    for i in range(3):
        for j in range(3):
            acc = m[i][0] * y[0][j] + y[i][0] * m[0][j]
            for kk in (1, 2):
                acc = acc + m[i][kk] * y[kk][j] + y[i][kk] * m[kk][j]
            c[i][j] = acc
    nrm = None
    for i in range(3):
        for j in range(3):
            t = c[i][j] * c[i][j]
            nrm = t if nrm is None else nrm + t
    inv = 1.0 / (nrm + 1.0)
    ivc = (c[0][0] + c[1][1] + c[2][2]) * (1.0 / 3.0)
    a01c = 0.5 * (c[0][1] - c[1][0])
    a02c = 0.5 * (c[0][2] - c[2][0])
    a12c = 0.5 * (c[1][2] - c[2][1])
    s00c = c[0][0] - ivc
    s11c = c[1][1] - ivc
    s01c = 0.5 * (c[0][1] + c[1][0])
    s02c = 0.5 * (c[0][2] + c[2][0])
    s12c = 0.5 * (c[1][2] + c[2][1])
    w3 = wt3[...]
    w4 = wt4[...]
    w5 = wt5[...]
    dot = lambda a, w: jnp.dot(a * inv, w, preferred_element_type=jnp.float32)
    D = _full9((dot(ivc, w3),
                dot(a01c, w4), dot(a02c, w4), dot(a12c, w4),
                dot(s00c, w5), dot(s01c, w5), dot(s02c, w5),
                dot(s11c, w5), dot(s12c, w5)))
    d = [[D[0], D[1], D[2]], [D[3], D[4], D[5]], [D[6], D[7], D[8]]]
    xn = xn_ref[...]
    for i in range(3):
        for j in range(3):
            acc = d[i][j]
            for kk in range(3):
                acc = acc + d[i][kk] * d[kk][j]
            out_ref[i * 3 + j] = xn[i * 3 + j] + acc


def kernel(X, edge_index, edge_weight, edge_attr,
           Ws1, bs1, Ws2, bs2, Ws3, bs3, Wt0, Wt1, Wt2, Wt3, Wt4, Wt5):
    f32 = jnp.float32
    X9 = X.reshape(_N, _H, 9).transpose(2, 0, 1)
    pad = _EP - _E
    ei = jnp.pad(edge_index.astype(jnp.int32), ((0, 0), (0, pad)))
    src_b = ei[0, :_TE].reshape(_TBLK, 1, _BEM)
    dst_b = ei[1, :_TE].reshape(_TBLK, 1, _BEM)
    idx3 = (ei[:, _TE:].reshape(2, _NW, _NBB, _BSC).transpose(1, 2, 0, 3))
    ea_p = jnp.pad(edge_attr, ((0, pad), (0, 0)))
    ew2 = jnp.pad(edge_weight, (0, pad),
                  constant_values=2.0 * _CUT).reshape(_EP, 1)
    b1 = bs1.reshape(1, _H)
    b2 = bs2.reshape(1, 2 * _H)
    w3s = [Ws3[:, ci::3] for ci in range(3)]
    b3s = [bs3[ci::3].reshape(1, _H) for ci in range(3)]

    full = lambda *shape: pl.BlockSpec(shape, lambda g: (0,) * len(shape))

    _BEP = 2048
    EA = pl.pallas_call(
        _edge_mlp_body,
        grid=(_EP // _BEP,),
        in_specs=[
            pl.BlockSpec((_BEP, _R), lambda g: (g, 0)),
            pl.BlockSpec((_BEP, 1), lambda g: (g, 0)),
            full(_R, _H), full(1, _H),
            full(_H, 2 * _H), full(1, 2 * _H),
            full(2 * _H, _H), full(1, _H),
            full(2 * _H, _H), full(1, _H),
            full(2 * _H, _H), full(1, _H),
        ],
        out_specs=pl.BlockSpec((3, _BEP, _H), lambda g: (0, g, 0)),
        out_shape=jax.ShapeDtypeStruct((3, _EP, _H), f32),
    )(ea_p, ew2, Ws1, b1, Ws2, b2,
      w3s[0], b3s[0], w3s[1], b3s[1], w3s[2], b3s[2])

    Xn9, V9, VA, VB = pl.pallas_call(
        _prep_body,
        grid=(_N // _BN,),
        in_specs=[
            pl.BlockSpec((9, _BN, _H), lambda g: (0, g, 0)),
            full(_H, _H), full(_H, _H), full(_H, _H),
        ],
        out_specs=[
            pl.BlockSpec((9, _BN, _H), lambda g: (0, g, 0)),
            pl.BlockSpec((9, _BN, _H), lambda g: (0, g, 0)),
            pl.BlockSpec((_BN, 4 * _H), lambda g: (g, 0)),
            pl.BlockSpec((_BN, 5 * _H), lambda g: (g, 0)),
        ],
        out_shape=[
            jax.ShapeDtypeStruct((9, _N, _H), f32),
            jax.ShapeDtypeStruct((9, _N, _H), f32),
            jax.ShapeDtypeStruct((_N, 4 * _H), f32),
            jax.ShapeDtypeStruct((_N, 5 * _H), f32),
        ],
    )(X9, Wt0, Wt1, Wt2)

    def _msg_pass(nk, ea_lo, ea_n, comp_of_k, V):
        return pl.pallas_call(
            _make_msg_body(comp_of_k),
            grid=(_TBLK,),
            in_specs=[
                pl.BlockSpec((1, 1, _BEM), lambda g: (g, 0, 0),
                             memory_space=pltpu.SMEM),
                pl.BlockSpec((1, 1, _BEM), lambda g: (g, 0, 0),
                             memory_space=pltpu.SMEM),
                pl.BlockSpec((ea_n, _BEM, _H), lambda g: (ea_lo, g, 0)),
                pl.BlockSpec((_N, nk * _H), lambda g: (0, 0)),
            ],
            out_specs=pl.BlockSpec((_N, nk * _H), lambda g: (0, 0)),
            out_shape=jax.ShapeDtypeStruct((_N, nk * _H), f32),
        )(src_b, dst_b, EA, V)

    MA = _msg_pass(4, 0, 2, (0, 1, 1, 1), VA)
    MB = _msg_pass(5, 2, 1, (0, 0, 0, 0, 0), VB)

    MSG2 = pl.kernel(
        _sc_msg_kernel,
        mesh=plsc.VectorSubcoreMesh(core_axis_name="c", subcore_axis_name="s"),
        out_type=jax.ShapeDtypeStruct((2, 9, _N, _H), f32),
        scratch_types=[
            pltpu.VMEM((2, _BSC), jnp.int32),
            pltpu.VMEM((2, _BSC), jnp.int32),
            pltpu.VMEM((_BSC, _H), f32),
            pltpu.VMEM((_BSC, _H), f32),
            pltpu.VMEM((_BSC, _H), f32),
            pltpu.VMEM((_BSC, _H), f32),
            pltpu.VMEM((_ZR, _H), f32),
            pltpu.VMEM_SHARED((_N, _H), f32),
            pltpu.SemaphoreType.DMA,
            pltpu.SemaphoreType.DMA,
            pltpu.SemaphoreType.DMA,
            pltpu.SemaphoreType.DMA,
            pltpu.SemaphoreType.DMA,
            pltpu.SemaphoreType.DMA,
        ],
    )(V9, EA, idx3)

    _BNP = 400
    OUT9 = pl.pallas_call(
        _post_body,
        grid=(_N // _BNP,),
        in_specs=[
            pl.BlockSpec((9, _BNP, _H), lambda g: (0, g, 0)),
            pl.BlockSpec((9, _BNP, _H), lambda g: (0, g, 0)),
            pl.BlockSpec((2, 9, _BNP, _H), lambda g: (0, 0, g, 0)),
            pl.BlockSpec((_BNP, 4 * _H), lambda g: (g, 0)),
            pl.BlockSpec((_BNP, 5 * _H), lambda g: (g, 0)),
            full(_H, _H), full(_H, _H), full(_H, _H),
        ],
        out_specs=pl.BlockSpec((9, _BNP, _H), lambda g: (0, g, 0)),
        out_shape=jax.ShapeDtypeStruct((9, _N, _H), f32),
    )(Xn9, V9, MSG2, MA, MB, Wt3, Wt4, Wt5)

    return OUT9.transpose(1, 2, 0).reshape(_N, _H, 3, 3)


# speedup vs baseline: 1.2131x; 1.2131x over previous
import jax
import jax.numpy as jnp
import numpy as np
from jax import lax
from jax.experimental import pallas as pl
from jax.experimental.pallas import tpu as pltpu
from jax.experimental.pallas import tpu_sc as plsc

_N = 10000
_E = 160000
_H = 128
_R = 32
_CUT = 5.0

_BN = 1000
_BE = 2000

_BEM = 2048
_TBLK = 43
_TE = _TBLK * _BEM
_NW = 32
_BSC = 32
_NBB = 72
_EPW = _BSC * _NBB
_EP = _TE + _NW * _EPW
_NPT = 624
_NTAIL = _N - 16 * _NPT
_ZR = 16
_G_OF_K = (0, 1, 1, 1, 2, 2, 2, 2, 2)


def _silu(x):
    return x / (1.0 + jnp.exp(-x))


def _edge_mlp_body(attr_ref, ew_ref, ws1, b1, ws2, b2,
                   w3a, b3a, w3b, b3b, w3c, b3c, out_ref):
    x = attr_ref[...]
    h1 = _silu(jnp.dot(x, ws1[...], preferred_element_type=jnp.float32) + b1[...])
    h2 = _silu(jnp.dot(h1, ws2[...], preferred_element_type=jnp.float32) + b2[...])
    r = ew_ref[...]
    c = 0.5 * (jnp.cos(r * (np.pi / _CUT)) + 1.0) * (r < _CUT).astype(jnp.float32)
    for ci, (w, b) in enumerate(((w3a, b3a), (w3b, b3b), (w3c, b3c))):
        out_ref[ci] = _silu(
            jnp.dot(h2, w[...], preferred_element_type=jnp.float32) + b[...]) * c


def _prep_body(x_ref, wt0, wt1, wt2, xn_ref, v_ref, va_ref, vb_ref):
    x = x_ref[...]
    nrm = (x * x).sum(axis=0)
    xn = x / (nrm + 1.0)
    xn_ref[...] = xn
    iv = (xn[0] + xn[4] + xn[8]) * (1.0 / 3.0)
    a01 = 0.5 * (xn[1] - xn[3])
    a02 = 0.5 * (xn[2] - xn[6])
    a12 = 0.5 * (xn[5] - xn[7])
    s00 = xn[0] - iv
    s11 = xn[4] - iv
    s01 = 0.5 * (xn[1] + xn[3])
    s02 = 0.5 * (xn[2] + xn[6])
    s12 = 0.5 * (xn[5] + xn[7])
    w0 = wt0[...]
    w1 = wt1[...]
    w2 = wt2[...]
    dot = lambda a, w: jnp.dot(a, w, preferred_element_type=jnp.float32)
    planes = (dot(iv, w0), dot(a01, w1), dot(a02, w1), dot(a12, w1),
              dot(s00, w2), dot(s01, w2), dot(s02, w2),
              dot(s11, w2), dot(s12, w2))
    for k in range(9):
        v_ref[k] = planes[k]
    for k in range(4):
        va_ref[:, k * _H:(k + 1) * _H] = planes[k]
    for k in range(5):
        vb_ref[:, k * _H:(k + 1) * _H] = planes[4 + k]


def _make_msg_body(comp_of_k):
    def _msg_body(src_ref, dst_ref, ea_ref, v_ref, msg_ref):
        @pl.when(pl.program_id(0) == 0)
        def _():
            msg_ref[...] = jnp.zeros_like(msg_ref)

        ncomp = len(sorted(set(comp_of_k)))

        def body(i, carry):
            s = src_ref[0, 0, i]
            d = dst_ref[0, 0, i]
            es = [ea_ref[c, pl.ds(i, 1), :] for c in range(ncomp)]
            m = jnp.concatenate([es[c] for c in comp_of_k], axis=1)
            msg_ref[pl.ds(d, 1), :] += v_ref[pl.ds(s, 1), :] * m
            return carry

        jax.lax.fori_loop(0, _BEM, body, 0, unroll=2)

    return _msg_body


def _sc_msg_kernel(v_hbm, ea_hbm, idx_hbm, out_hbm,
                   idxb0, idxb1, rows0, rows1, eav0, eav1, zrow, shared,
                   si0, si1, sg0, sg1, se0, se1):
    cid = lax.axis_index("c")
    sid = lax.axis_index("s")
    wid = cid * 16 + sid
    edge_base = _TE + wid * _EPW
    row_lo = sid * _NPT

    def zinit(i, c):
        for j in range(_H // 16):
            zrow[i, pl.ds(j * 16, 16)] = jnp.zeros((16,), jnp.float32)
        return c

    lax.fori_loop(0, _ZR, zinit, 0)

    def start_idx(b, idxb, si):
        pltpu.async_copy(idx_hbm.at[wid, b], idxb, si)

    def wait_idx(b, idxb, si):
        pltpu.make_async_copy(idx_hbm.at[wid, b], idxb, si).wait()

    for k in range(9):
        gk = _G_OF_K[k]

        def start_ge(b, idxb, rows, eav, sg, se):
            base = pl.multiple_of(edge_base + b * _BSC, _BSC)
            pltpu.async_copy(v_hbm.at[k].at[idxb.at[0]], rows, sg)
            pltpu.async_copy(ea_hbm.at[gk, pl.ds(base, _BSC)], eav, se)

        def finish(b, idxb, rows, eav, sg, se):
            pltpu.make_async_copy(v_hbm.at[k].at[idxb.at[0]],
                                  rows, sg).wait()
            base = pl.multiple_of(edge_base + b * _BSC, _BSC)
            pltpu.make_async_copy(ea_hbm.at[gk, pl.ds(base, _BSC)],
                                  eav, se).wait()

            def edge(i, c2):
                for j in range(_H // 16):
                    rows[i, pl.ds(j * 16, 16)] = (
                        rows[i, pl.ds(j * 16, 16)]
                        * eav[i, pl.ds(j * 16, 16)])
                return c2

            lax.fori_loop(0, _BSC, edge, 0, unroll=2)
            pltpu.sync_copy(rows, shared.at[idxb.at[1]], add=True)

        for z in range(_NPT // _ZR):
            pltpu.sync_copy(zrow, shared.at[pl.ds(row_lo + z * _ZR, _ZR)])

        @pl.when(sid == 15)
        def _():
            pltpu.sync_copy(zrow.at[pl.ds(0, _NTAIL)],
                            shared.at[pl.ds(16 * _NPT, _NTAIL)])

        plsc.subcore_barrier()
        pltpu.sync_copy(idx_hbm.at[wid, 0], idxb0)
        start_ge(0, idxb0, rows0, eav0, sg0, se0)
        start_idx(1, idxb1, si1)

        def pair(p, carry):
            b0 = p * 2
            wait_idx(b0 + 1, idxb1, si1)
            start_ge(b0 + 1, idxb1, rows1, eav1, sg1, se1)
            finish(b0, idxb0, rows0, eav0, sg0, se0)

            @pl.when(b0 + 2 < _NBB)
            def _():
                start_idx(b0 + 2, idxb0, si0)

            finish(b0 + 1, idxb1, rows1, eav1, sg1, se1)

            @pl.when(b0 + 3 < _NBB)
            def _():
                start_idx(b0 + 3, idxb1, si1)

            @pl.when(b0 + 2 < _NBB)
            def _():
                wait_idx(b0 + 2, idxb0, si0)
                start_ge(b0 + 2, idxb0, rows0, eav0, sg0, se0)

            return carry

        lax.fori_loop(0, _NBB // 2, pair, 0)
        plsc.subcore_barrier()
        pltpu.sync_copy(shared.at[pl.ds(row_lo, _NPT)],
                        out_hbm.at[cid, k, pl.ds(row_lo, _NPT)])

        @pl.when(sid == 15)
        def _():
            pltpu.sync_copy(shared.at[pl.ds(16 * _NPT, _NTAIL)],
                            out_hbm.at[cid, k, pl.ds(16 * _NPT, _NTAIL)])

        plsc.subcore_barrier()


def _full9(t):
    iv, a01, a02, a12, s00, s01, s02, s11, s12 = t
    return (iv + s00, s01 + a01, s02 + a02,
            s01 - a01, iv + s11, s12 + a12,
            s02 - a02, s12 - a12, iv - s00 - s11)


def _post_body(xn_ref, v_ref, m_ref, ma_ref, mb_ref, wt3, wt4, wt5, out_ref):
    vv = v_ref[...]
    vm = m_ref[0] + m_ref[1]
    Y = _full9(tuple(vv[k] for k in range(9)))
    M = _full9(tuple(
        vm[k] + (ma_ref[:, k * _H:(k + 1) * _H] if k < 4
                 else mb_ref[:, (k - 4) * _H:(k - 3) * _H])
        for k in range(9)))
    y = [[Y[0], Y[1], Y[2]], [Y[3], Y[4], Y[5]], [Y[6], Y[7], Y[8]]]
    m = [[M[0], M[1], M[2]], [M[3], M[4], M[5]], [M[6], M[7], M[8]]]
    c = [[None] * 3 for _ in range(3)]
    for i in range(3):
        for j in range(3):
            acc = m[i][0] * y[0][j] + y[i][0] * m[0][j]
            for kk in (1, 2):
                acc = acc + m[i][kk] * y[kk][j] + y[i][kk] * m[kk][j]
            c[i][j] = acc
    nrm = None
    for i in range(3):
        for j in range(3):
            t = c[i][j] * c[i][j]
            nrm = t if nrm is None else nrm + t
    inv = 1.0 / (nrm + 1.0)
    ivc = (c[0][0] + c[1][1] + c[2][2]) * (1.0 / 3.0)
    a01c = 0.5 * (c[0][1] - c[1][0])
    a02c = 0.5 * (c[0][2] - c[2][0])
    a12c = 0.5 * (c[1][2] - c[2][1])
    s00c = c[0][0] - ivc
    s11c = c[1][1] - ivc
    s01c = 0.5 * (c[0][1] + c[1][0])
    s02c = 0.5 * (c[0][2] + c[2][0])
    s12c = 0.5 * (c[1][2] + c[2][1])
    w3 = wt3[...]
    w4 = wt4[...]
    w5 = wt5[...]
    dot = lambda a, w: jnp.dot(a * inv, w, preferred_element_type=jnp.float32)
    D = _full9((dot(ivc, w3),
                dot(a01c, w4), dot(a02c, w4), dot(a12c, w4),
                dot(s00c, w5), dot(s01c, w5), dot(s02c, w5),
                dot(s11c, w5), dot(s12c, w5)))
    d = [[D[0], D[1], D[2]], [D[3], D[4], D[5]], [D[6], D[7], D[8]]]
    xn = xn_ref[...]
    for i in range(3):
        for j in range(3):
            acc = d[i][j]
            for kk in range(3):
                acc = acc + d[i][kk] * d[kk][j]
            out_ref[i * 3 + j] = xn[i * 3 + j] + acc


def kernel(X, edge_index, edge_weight, edge_attr,
           Ws1, bs1, Ws2, bs2, Ws3, bs3, Wt0, Wt1, Wt2, Wt3, Wt4, Wt5):
    f32 = jnp.float32
    X9 = X.reshape(_N, _H, 9).transpose(2, 0, 1)
    pad = _EP - _E
    ei = jnp.pad(edge_index.astype(jnp.int32), ((0, 0), (0, pad)))
    src_b = ei[0, :_TE].reshape(_TBLK, 1, _BEM)
    dst_b = ei[1, :_TE].reshape(_TBLK, 1, _BEM)
    idx3 = (ei[:, _TE:].reshape(2, _NW, _NBB, _BSC).transpose(1, 2, 0, 3))
    ea_p = jnp.pad(edge_attr, ((0, pad), (0, 0)))
    ew2 = jnp.pad(edge_weight, (0, pad),
                  constant_values=2.0 * _CUT).reshape(_EP, 1)
    b1 = bs1.reshape(1, _H)
    b2 = bs2.reshape(1, 2 * _H)
    w3s = [Ws3[:, ci::3] for ci in range(3)]
    b3s = [bs3[ci::3].reshape(1, _H) for ci in range(3)]

    full = lambda *shape: pl.BlockSpec(shape, lambda g: (0,) * len(shape))

    _BEP = 2048
    EA = pl.pallas_call(
        _edge_mlp_body,
        grid=(_EP // _BEP,),
        in_specs=[
            pl.BlockSpec((_BEP, _R), lambda g: (g, 0)),
            pl.BlockSpec((_BEP, 1), lambda g: (g, 0)),
            full(_R, _H), full(1, _H),
            full(_H, 2 * _H), full(1, 2 * _H),
            full(2 * _H, _H), full(1, _H),
            full(2 * _H, _H), full(1, _H),
            full(2 * _H, _H), full(1, _H),
        ],
        out_specs=pl.BlockSpec((3, _BEP, _H), lambda g: (0, g, 0)),
        out_shape=jax.ShapeDtypeStruct((3, _EP, _H), f32),
    )(ea_p, ew2, Ws1, b1, Ws2, b2,
      w3s[0], b3s[0], w3s[1], b3s[1], w3s[2], b3s[2])

    Xn9, V9, VA, VB = pl.pallas_call(
        _prep_body,
        grid=(_N // _BN,),
        in_specs=[
            pl.BlockSpec((9, _BN, _H), lambda g: (0, g, 0)),
            full(_H, _H), full(_H, _H), full(_H, _H),
        ],
        out_specs=[
            pl.BlockSpec((9, _BN, _H), lambda g: (0, g, 0)),
            pl.BlockSpec((9, _BN, _H), lambda g: (0, g, 0)),
            pl.BlockSpec((_BN, 4 * _H), lambda g: (g, 0)),
            pl.BlockSpec((_BN, 5 * _H), lambda g: (g, 0)),
        ],
        out_shape=[
            jax.ShapeDtypeStruct((9, _N, _H), f32),
            jax.ShapeDtypeStruct((9, _N, _H), f32),
            jax.ShapeDtypeStruct((_N, 4 * _H), f32),
            jax.ShapeDtypeStruct((_N, 5 * _H), f32),
        ],
    )(X9, Wt0, Wt1, Wt2)

    def _msg_pass(nk, ea_lo, ea_n, comp_of_k, V):
        return pl.pallas_call(
            _make_msg_body(comp_of_k),
            grid=(_TBLK,),
            in_specs=[
                pl.BlockSpec((1, 1, _BEM), lambda g: (g, 0, 0),
                             memory_space=pltpu.SMEM),
                pl.BlockSpec((1, 1, _BEM), lambda g: (g, 0, 0),
                             memory_space=pltpu.SMEM),
                pl.BlockSpec((ea_n, _BEM, _H), lambda g: (ea_lo, g, 0)),
                pl.BlockSpec((_N, nk * _H), lambda g: (0, 0)),
            ],
            out_specs=pl.BlockSpec((_N, nk * _H), lambda g: (0, 0)),
            out_shape=jax.ShapeDtypeStruct((_N, nk * _H), f32),
        )(src_b, dst_b, EA, V)

    MA = _msg_pass(4, 0, 2, (0, 1, 1, 1), VA)
    MB = _msg_pass(5, 2, 1, (0, 0, 0, 0, 0), VB)

    MSG2 = pl.kernel(
        _sc_msg_kernel,
        mesh=plsc.VectorSubcoreMesh(core_axis_name="c", subcore_axis_name="s"),
        out_type=jax.ShapeDtypeStruct((2, 9, _N, _H), f32),
        scratch_types=[
            pltpu.VMEM((2, _BSC), jnp.int32),
            pltpu.VMEM((2, _BSC), jnp.int32),
            pltpu.VMEM((_BSC, _H), f32),
            pltpu.VMEM((_BSC, _H), f32),
            pltpu.VMEM((_BSC, _H), f32),
            pltpu.VMEM((_BSC, _H), f32),
            pltpu.VMEM((_ZR, _H), f32),
            pltpu.VMEM_SHARED((_N, _H), f32),
            pltpu.SemaphoreType.DMA,
            pltpu.SemaphoreType.DMA,
            pltpu.SemaphoreType.DMA,
            pltpu.SemaphoreType.DMA,
            pltpu.SemaphoreType.DMA,
            pltpu.SemaphoreType.DMA,
        ],
    )(V9, EA, idx3)

    _BNP = 400
    OUT9 = pl.pallas_call(
        _post_body,
        grid=(_N // _BNP,),
        in_specs=[
            pl.BlockSpec((9, _BNP, _H), lambda g: (0, g, 0)),
            pl.BlockSpec((9, _BNP, _H), lambda g: (0, g, 0)),
            pl.BlockSpec((2, 9, _BNP, _H), lambda g: (0, 0, g, 0)),
            pl.BlockSpec((_BNP, 4 * _H), lambda g: (g, 0)),
            pl.BlockSpec((_BNP, 5 * _H), lambda g: (g, 0)),
            full(_H, _H), full(_H, _H), full(_H, _H),
        ],
        out_specs=pl.BlockSpec((9, _BNP, _H), lambda g: (0, g, 0)),
        out_shape=jax.ShapeDtypeStruct((9, _N, _H), f32),
    )(Xn9, V9, MSG2, MA, MB, Wt3, Wt4, Wt5)

    return OUT9.transpose(1, 2, 0).reshape(_N, _H, 3, 3)
